# baseline (speedup 1.0000x reference)
import functools

import jax
import jax.numpy as jnp
from jax import lax
from jax.experimental import pallas as pl
from jax.experimental.pallas import tpu as pltpu
from jax.experimental.pallas import tpu_sc as plsc

NF = 384
B = 32
HW = 1024
CH = 128
NCH = NF // CH

NWORK = 4
RPW = B // NWORK


def _make_gather():
    mesh = plsc.VectorSubcoreMesh(core_axis_name="c", subcore_axis_name="s")

    @functools.partial(
        pl.kernel,
        mesh=mesh,
        out_type=jax.ShapeDtypeStruct((B, 2 * NF), jnp.float32),
        scratch_types=[
            pltpu.VMEM((RPW,), jnp.int32),
            pltpu.VMEM((RPW, 2 * NF), jnp.float32),
            pltpu.SemaphoreType.DMA,
        ],
    )
    def gather(table_hbm, idx_hbm, out_hbm, idx_v, rows_v, sem):
        wid = lax.axis_index("s") * 2 + lax.axis_index("c")

        @pl.when(wid < NWORK)
        def _():
            base = wid * RPW
            pltpu.sync_copy(idx_hbm.at[pl.ds(base, RPW)], idx_v)
            pltpu.async_copy(table_hbm.at[idx_v], rows_v, sem).wait()
            pltpu.sync_copy(rows_v, out_hbm.at[pl.ds(base, RPW)])

    return gather


_gather = _make_gather()


N_BUF = 8
NCHUNK = B * NCH


def _affine_body(e_ref, x_ref, o_ref, in_buf, out_buf, in_sem, out_sem):
    def in_start(i, slot):
        row = pl.multiple_of(i * CH, 128)
        pltpu.make_async_copy(
            x_ref.at[pl.ds(row, CH)], in_buf.at[slot], in_sem.at[slot]
        ).start()

    def out_copy(i, slot):
        row = pl.multiple_of(i * CH, 128)
        return pltpu.make_async_copy(
            out_buf.at[slot], o_ref.at[pl.ds(row, CH)], out_sem.at[slot]
        )

    for k in range(N_BUF):
        in_start(k, k)

    def body(i, carry):
        slot = lax.rem(i, N_BUF)
        bi = lax.div(i, NCH)
        j = lax.rem(i, NCH)
        row = pl.multiple_of(i * CH, 128)
        pltpu.make_async_copy(
            x_ref.at[pl.ds(row, CH)], in_buf.at[slot], in_sem.at[slot]
        ).wait()

        @pl.when(i >= N_BUF)
        def _():
            out_copy(i - N_BUF, slot).wait()

        off = pl.multiple_of(j * CH, 128)
        g = e_ref[bi, 0, pl.ds(off, CH)].reshape(CH, 1)
        bb = e_ref[bi, 0, pl.ds(NF + off, CH)].reshape(CH, 1)
        out_buf[slot] = in_buf[slot] * g + bb
        out_copy(i, slot).start()

        @pl.when(i + N_BUF < NCHUNK)
        def _():
            in_start(i + N_BUF, slot)

        return carry

    lax.fori_loop(0, NCHUNK, body, 0)
    for k in range(N_BUF):
        chunk = NCHUNK - N_BUF + k
        out_copy(chunk, chunk % N_BUF).wait()


def kernel(x, y, embed_weight):
    y32 = y.astype(jnp.int32)
    e = _gather(embed_weight, y32)
    xf = x.reshape(B * NF, HW)
    out = pl.pallas_call(
        _affine_body,
        in_specs=[
            pl.BlockSpec((B, 1, 2 * NF), lambda: (0, 0, 0)),
            pl.BlockSpec(memory_space=pl.ANY),
        ],
        out_specs=pl.BlockSpec(memory_space=pl.ANY),
        out_shape=jax.ShapeDtypeStruct((B * NF, HW), jnp.float32),
        scratch_shapes=[
            pltpu.VMEM((N_BUF, CH, HW), jnp.float32),
            pltpu.VMEM((N_BUF, CH, HW), jnp.float32),
            pltpu.SemaphoreType.DMA((N_BUF,)),
            pltpu.SemaphoreType.DMA((N_BUF,)),
        ],
    )(e.reshape(B, 1, 2 * NF), xf)
    return out.reshape(x.shape)


# speedup vs baseline: 4.7793x; 4.7793x over previous
import functools

import jax
import jax.numpy as jnp
from jax import lax
from jax.experimental import pallas as pl
from jax.experimental.pallas import tpu as pltpu
from jax.experimental.pallas import tpu_sc as plsc

NF = 384
B = 32
HW = 1024
CH = 128
NCH = NF // CH

NWORK = 4
RPW = B // NWORK


def _make_gather():
    mesh = plsc.VectorSubcoreMesh(core_axis_name="c", subcore_axis_name="s")

    @functools.partial(
        pl.kernel,
        mesh=mesh,
        out_type=jax.ShapeDtypeStruct((B, 2 * NF), jnp.float32),
        scratch_types=[
            pltpu.VMEM((RPW,), jnp.int32),
            pltpu.VMEM((RPW, 2 * NF), jnp.float32),
            pltpu.SemaphoreType.DMA,
        ],
    )
    def gather(table_hbm, idx_hbm, out_hbm, idx_v, rows_v, sem):
        wid = lax.axis_index("s") * 2 + lax.axis_index("c")

        @pl.when(wid < NWORK)
        def _():
            base = wid * RPW
            pltpu.sync_copy(idx_hbm.at[pl.ds(base, RPW)], idx_v)
            pltpu.async_copy(table_hbm.at[idx_v], rows_v, sem).wait()
            pltpu.sync_copy(rows_v, out_hbm.at[pl.ds(base, RPW)])

    return gather


_gather = _make_gather()


BLK = 1024
BPB = HW // BLK


def _affine_body(e_ref, x_ref, o_ref):
    g = e_ref[0, 0, :NF]
    b = e_ref[0, 0, NF:]
    o_ref[...] = x_ref[...] * g + b


def kernel(x, y, embed_weight):
    y32 = y.astype(jnp.int32)
    e = _gather(embed_weight, y32)
    H, W = x.shape[2], x.shape[3]
    xf = x.transpose(0, 2, 3, 1).reshape(B * HW, NF)
    out = pl.pallas_call(
        _affine_body,
        grid=(B * HW // BLK,),
        in_specs=[
            pl.BlockSpec((1, 1, 2 * NF), lambda i: (i // BPB, 0, 0)),
            pl.BlockSpec((BLK, NF), lambda i: (i, 0)),
        ],
        out_specs=pl.BlockSpec((BLK, NF), lambda i: (i, 0)),
        out_shape=jax.ShapeDtypeStruct((B * HW, NF), jnp.float32),
    )(e.reshape(B, 1, 2 * NF), xf)
    return out.reshape(B, H, W, NF).transpose(0, 3, 1, 2)


# speedup vs baseline: 6.3278x; 1.3240x over previous
import functools

import jax
import jax.numpy as jnp
from jax import lax
from jax.experimental import pallas as pl
from jax.experimental.pallas import tpu as pltpu
from jax.experimental.pallas import tpu_sc as plsc

NF = 384
B = 32
HW = 1024
CH = 128
NCH = NF // CH

NWORK = 4
RPW = B // NWORK


def _make_gather():
    mesh = plsc.VectorSubcoreMesh(core_axis_name="c", subcore_axis_name="s")

    @functools.partial(
        pl.kernel,
        mesh=mesh,
        out_type=jax.ShapeDtypeStruct((B, 2 * NF), jnp.float32),
        scratch_types=[
            pltpu.VMEM((RPW,), jnp.int32),
            pltpu.VMEM((RPW, 2 * NF), jnp.float32),
            pltpu.SemaphoreType.DMA,
        ],
    )
    def gather(table_hbm, idx_hbm, out_hbm, idx_v, rows_v, sem):
        wid = lax.axis_index("s") * 2 + lax.axis_index("c")

        @pl.when(wid < NWORK)
        def _():
            base = wid * RPW
            pltpu.sync_copy(idx_hbm.at[pl.ds(base, RPW)], idx_v)
            pltpu.async_copy(table_hbm.at[idx_v], rows_v, sem).wait()
            pltpu.sync_copy(rows_v, out_hbm.at[pl.ds(base, RPW)])

    return gather


_gather = _make_gather()


BLK = 1024
BPB = HW // BLK


def _affine_body(y_ref, e_ref, x_ref, o_ref):
    g = e_ref[0, 0, :NF]
    b = e_ref[0, 0, NF:]
    o_ref[...] = x_ref[...] * g + b


def kernel(x, y, embed_weight):
    y32 = y.astype(jnp.int32)
    H, W = x.shape[2], x.shape[3]
    xf = x.transpose(0, 2, 3, 1).reshape(B * HW, NF)
    grid_spec = pltpu.PrefetchScalarGridSpec(
        num_scalar_prefetch=1,
        grid=(B * HW // BLK,),
        in_specs=[
            pl.BlockSpec((1, 1, 2 * NF), lambda i, yv: (yv[i // BPB], 0, 0)),
            pl.BlockSpec((BLK, NF), lambda i, yv: (i, 0)),
        ],
        out_specs=pl.BlockSpec((BLK, NF), lambda i, yv: (i, 0)),
    )
    out = pl.pallas_call(
        _affine_body,
        grid_spec=grid_spec,
        out_shape=jax.ShapeDtypeStruct((B * HW, NF), jnp.float32),
    )(y32, embed_weight.reshape(-1, 1, 2 * NF), xf)
    return out.reshape(B, H, W, NF).transpose(0, 3, 1, 2)
